# baseline (speedup 1.0000x reference)
import functools

import jax
import jax.numpy as jnp
from jax import lax
from jax.experimental import pallas as pl
from jax.experimental.pallas import tpu as pltpu
from jax.experimental.pallas import tpu_sc as plsc

B = 4096
F = 26
V = 1000
D = 13
W_LEN = F * V + D

NC = 2
NS = 16
NW = NC * NS
BPW = B // NW
L = 16
G = BPW // L
NIDX = F * BPW


def _wide_sc(idx_blocks, dense_blocks, w_flat, w_head):
    mesh = plsc.VectorSubcoreMesh(core_axis_name="c", subcore_axis_name="s")

    @functools.partial(
        pl.kernel,
        mesh=mesh,
        out_type=jax.ShapeDtypeStruct((B,), jnp.float32),
        compiler_params=pltpu.CompilerParams(needs_layout_passes=False),
        scratch_types=[
            pltpu.VMEM((NIDX,), jnp.int32),
            pltpu.VMEM((D, BPW), jnp.float32),
            pltpu.VMEM((L,), jnp.float32),
            pltpu.VMEM((NIDX,), jnp.float32),
            pltpu.VMEM((BPW,), jnp.float32),
            pltpu.SemaphoreType.DMA,
            pltpu.SemaphoreType.DMA,
        ],
    )
    def body(idx_hbm, dense_hbm, w_hbm, whead_hbm, out_hbm,
             idx_v, dense_v, whead_v, gath_v, out_v, sem, gsem):
        wid = lax.axis_index("s") * NC + lax.axis_index("c")
        c_idx = pltpu.async_copy(idx_hbm.at[wid], idx_v, sem)
        c_dense = pltpu.async_copy(dense_hbm.at[wid], dense_v, sem)
        c_head = pltpu.async_copy(whead_hbm, whead_v, sem)
        c_idx.wait()
        c_gath = pltpu.async_copy(w_hbm.at[idx_v], gath_v, gsem)
        c_dense.wait()
        c_head.wait()
        w016 = whead_v[...]
        for g in range(G):
            sl = pl.ds(g * L, L)
            acc = dense_v[0, sl] * w016[0]
            for d in range(1, D):
                acc = acc + dense_v[d, sl] * w016[d]
            out_v[sl] = acc
        c_gath.wait()
        for g in range(G):
            sl = pl.ds(g * L, L)
            acc = out_v[sl]
            for i in range(F):
                acc = acc + gath_v[pl.ds(i * BPW + g * L, L)]
            out_v[sl] = acc
        pltpu.sync_copy(out_v, out_hbm.at[pl.ds(wid * BPW, BPW)])

    return body(idx_blocks, dense_blocks, w_flat, w_head)


def kernel(dense_features, W,
           sparse_0, sparse_1, sparse_2, sparse_3, sparse_4, sparse_5,
           sparse_6, sparse_7, sparse_8, sparse_9, sparse_10, sparse_11,
           sparse_12, sparse_13, sparse_14, sparse_15, sparse_16, sparse_17,
           sparse_18, sparse_19, sparse_20, sparse_21, sparse_22, sparse_23,
           sparse_24, sparse_25):
    sparse = [sparse_0, sparse_1, sparse_2, sparse_3, sparse_4, sparse_5,
              sparse_6, sparse_7, sparse_8, sparse_9, sparse_10, sparse_11,
              sparse_12, sparse_13, sparse_14, sparse_15, sparse_16,
              sparse_17, sparse_18, sparse_19, sparse_20, sparse_21,
              sparse_22, sparse_23, sparse_24, sparse_25]
    offs = (D + V * jnp.arange(F, dtype=jnp.int32))[:, None]
    idx = jnp.stack(sparse, axis=0) + offs
    idx_blocks = (idx.reshape(F, NW, BPW).transpose(1, 0, 2)
                  .reshape(NW, NIDX))
    dense_blocks = dense_features.T.reshape(D, NW, BPW).transpose(1, 0, 2)
    w_flat = W[:, 0]
    w_head = w_flat[:L]
    out = _wide_sc(idx_blocks, dense_blocks, w_flat, w_head)
    return out[:, None]


# speedup vs baseline: 1.2151x; 1.2151x over previous
import functools

import jax
import jax.numpy as jnp
from jax import lax
from jax.experimental import pallas as pl
from jax.experimental.pallas import tpu as pltpu
from jax.experimental.pallas import tpu_sc as plsc

B = 4096
F = 26
V = 1000
D = 13
W_LEN = F * V + D

NC = 2
NS = 16
NW = NC * NS
BPW = B // NW
L = 16
G = BPW // L


def _wide_sc(dense_features, w_flat, sparse):
    mesh = plsc.VectorSubcoreMesh(core_axis_name="c", subcore_axis_name="s")

    @functools.partial(
        pl.kernel,
        mesh=mesh,
        out_type=jax.ShapeDtypeStruct((B,), jnp.float32),
        compiler_params=pltpu.CompilerParams(needs_layout_passes=False),
        scratch_types=[
            pltpu.VMEM((F, BPW), jnp.int32),
            pltpu.VMEM((BPW, D), jnp.float32),
            pltpu.VMEM((W_LEN,), jnp.float32),
            pltpu.VMEM((BPW,), jnp.float32),
            pltpu.SemaphoreType.DMA,
        ],
    )
    def body(dense_hbm, w_hbm, *refs):
        sparse_hbm = refs[:F]
        out_hbm, idx_v, dense_v, w_v, out_v, sem = refs[F:]
        wid = lax.axis_index("s") * NC + lax.axis_index("c")
        rows = pl.ds(wid * BPW, BPW)
        copies = [pltpu.async_copy(w_hbm, w_v, sem),
                  pltpu.async_copy(dense_hbm.at[rows], dense_v, sem)]
        for i in range(F):
            copies.append(
                pltpu.async_copy(sparse_hbm[i].at[rows], idx_v.at[i], sem))
        for c in copies:
            c.wait()
        w016 = w_v[pl.ds(0, L)]
        zero16 = lax.iota(jnp.int32, L) * 0
        for g in range(G):
            sl = pl.ds(g * L, L)
            rvec = lax.iota(jnp.int32, L) + (g * L)
            acc = plsc.load_gather(dense_v, [rvec, zero16]) * w016[0]
            for d in range(1, D):
                acc = acc + (plsc.load_gather(dense_v, [rvec, zero16 + d])
                             * w016[d])
            for i in range(F):
                gidx = idx_v[i, sl] + (D + i * V)
                acc = acc + plsc.load_gather(w_v, [gidx])
            out_v[sl] = acc
        pltpu.sync_copy(out_v, out_hbm.at[rows])

    return body(dense_features, w_flat, *sparse)


def kernel(dense_features, W,
           sparse_0, sparse_1, sparse_2, sparse_3, sparse_4, sparse_5,
           sparse_6, sparse_7, sparse_8, sparse_9, sparse_10, sparse_11,
           sparse_12, sparse_13, sparse_14, sparse_15, sparse_16, sparse_17,
           sparse_18, sparse_19, sparse_20, sparse_21, sparse_22, sparse_23,
           sparse_24, sparse_25):
    sparse = [sparse_0, sparse_1, sparse_2, sparse_3, sparse_4, sparse_5,
              sparse_6, sparse_7, sparse_8, sparse_9, sparse_10, sparse_11,
              sparse_12, sparse_13, sparse_14, sparse_15, sparse_16,
              sparse_17, sparse_18, sparse_19, sparse_20, sparse_21,
              sparse_22, sparse_23, sparse_24, sparse_25]
    out = _wide_sc(dense_features, W[:, 0], sparse)
    return out[:, None]


# speedup vs baseline: 1.3836x; 1.1387x over previous
import functools

import jax
import jax.numpy as jnp
from jax import lax
from jax.experimental import pallas as pl
from jax.experimental.pallas import tpu as pltpu
from jax.experimental.pallas import tpu_sc as plsc

B = 4096
F = 26
V = 1000
D = 13
W_LEN = F * V + D

NC = 2
NS = 16
NW = NC * NS
BPW = B // NW
L = 16
G = BPW // L
NIDX = F * BPW
NSLC = 1632
W_IMG = NS * NSLC


def _wide_sc(dense_features, w_flat, sparse):
    mesh = plsc.VectorSubcoreMesh(core_axis_name="c", subcore_axis_name="s")

    @functools.partial(
        pl.kernel,
        mesh=mesh,
        out_type=jax.ShapeDtypeStruct((B,), jnp.float32),
        compiler_params=pltpu.CompilerParams(needs_layout_passes=False),
        scratch_types=[
            pltpu.VMEM((NIDX,), jnp.int32),
            pltpu.VMEM((BPW, D), jnp.float32),
            pltpu.VMEM((L,), jnp.float32),
            pltpu.VMEM((NIDX,), jnp.float32),
            pltpu.VMEM((BPW,), jnp.float32),
            pltpu.VMEM((NSLC,), jnp.float32),
            pltpu.VMEM_SHARED((W_IMG,), jnp.float32),
            pltpu.SemaphoreType.DMA,
            pltpu.SemaphoreType.DMA,
        ],
    )
    def body(dense_hbm, w_hbm, *refs):
        sparse_hbm = refs[:F]
        (out_hbm, idx_v, dense_v, whead_v, gath_v, out_v, w_stage_v, w_sh,
         sem, gsem) = refs[F:]
        cid = lax.axis_index("c")
        sid = lax.axis_index("s")
        wid = sid * NC + cid
        rows = pl.ds(wid * BPW, BPW)
        copies = [pltpu.async_copy(dense_hbm.at[rows], dense_v, sem),
                  pltpu.async_copy(w_hbm.at[pl.ds(0, L)], whead_v, sem)]
        for i in range(F):
            copies.append(pltpu.async_copy(
                sparse_hbm[i].at[rows], idx_v.at[pl.ds(i * BPW, BPW)], sem))
        for k in range(NS):
            size = min(NSLC, W_LEN - k * NSLC)
            @pl.when(sid == k)
            def _():
                pltpu.sync_copy(w_hbm.at[pl.ds(k * NSLC, size)],
                                w_stage_v.at[pl.ds(0, size)])
                pltpu.sync_copy(w_stage_v.at[pl.ds(0, size)],
                                w_sh.at[pl.ds(k * NSLC, size)])
        plsc.subcore_barrier()
        for c in copies[2:]:
            c.wait()
        for i in range(F):
            off = D + i * V
            for g in range(G):
                sl = pl.ds(i * BPW + g * L, L)
                idx_v[sl] = idx_v[sl] + off
        c_gath = pltpu.async_copy(w_sh.at[idx_v], gath_v, gsem)
        copies[0].wait()
        copies[1].wait()
        w016 = whead_v[...]
        zero16 = lax.iota(jnp.int32, L) * 0
        for g in range(G):
            sl = pl.ds(g * L, L)
            rvec = lax.iota(jnp.int32, L) + (g * L)
            acc = plsc.load_gather(dense_v, [rvec, zero16]) * w016[0]
            for d in range(1, D):
                acc = acc + (plsc.load_gather(dense_v, [rvec, zero16 + d])
                             * w016[d])
            out_v[sl] = acc
        c_gath.wait()
        for g in range(G):
            sl = pl.ds(g * L, L)
            acc = out_v[sl]
            for i in range(F):
                acc = acc + gath_v[pl.ds(i * BPW + g * L, L)]
            out_v[sl] = acc
        pltpu.sync_copy(out_v, out_hbm.at[rows])

    return body(dense_features, w_flat, *sparse)


def kernel(dense_features, W,
           sparse_0, sparse_1, sparse_2, sparse_3, sparse_4, sparse_5,
           sparse_6, sparse_7, sparse_8, sparse_9, sparse_10, sparse_11,
           sparse_12, sparse_13, sparse_14, sparse_15, sparse_16, sparse_17,
           sparse_18, sparse_19, sparse_20, sparse_21, sparse_22, sparse_23,
           sparse_24, sparse_25):
    sparse = [sparse_0, sparse_1, sparse_2, sparse_3, sparse_4, sparse_5,
              sparse_6, sparse_7, sparse_8, sparse_9, sparse_10, sparse_11,
              sparse_12, sparse_13, sparse_14, sparse_15, sparse_16,
              sparse_17, sparse_18, sparse_19, sparse_20, sparse_21,
              sparse_22, sparse_23, sparse_24, sparse_25]
    out = _wide_sc(dense_features, W[:, 0], sparse)
    return out[:, None]


# speedup vs baseline: 1.4134x; 1.0216x over previous
import functools

import jax
import jax.numpy as jnp
from jax import lax
from jax.experimental import pallas as pl
from jax.experimental.pallas import tpu as pltpu
from jax.experimental.pallas import tpu_sc as plsc

B = 4096
F = 26
V = 1000
D = 13
W_LEN = F * V + D

NC = 2
NS = 16
NW = NC * NS
BPW = B // NW
L = 16
G = BPW // L
NIDX = F * BPW
NSLC = 1632
W_IMG = NS * NSLC


def _wide_sc(dense_features, w_flat, sparse):
    mesh = plsc.VectorSubcoreMesh(core_axis_name="c", subcore_axis_name="s")

    @functools.partial(
        pl.kernel,
        mesh=mesh,
        out_type=jax.ShapeDtypeStruct((B,), jnp.float32),
        compiler_params=pltpu.CompilerParams(needs_layout_passes=False),
        scratch_types=[
            pltpu.VMEM((NIDX,), jnp.int32),
            pltpu.VMEM((BPW, D), jnp.float32),
            pltpu.VMEM((L,), jnp.float32),
            pltpu.VMEM((NIDX,), jnp.float32),
            pltpu.VMEM((BPW,), jnp.float32),
            pltpu.VMEM((NSLC,), jnp.float32),
            pltpu.VMEM_SHARED((W_IMG,), jnp.float32),
            pltpu.SemaphoreType.DMA,
            pltpu.SemaphoreType.DMA,
            pltpu.SemaphoreType.DMA,
            pltpu.SemaphoreType.DMA,
            pltpu.SemaphoreType.DMA,
        ],
    )
    def body(dense_hbm, w_hbm, *refs):
        sparse_hbm = refs[:F]
        (out_hbm, idx_v, dense_v, whead_v, gath_v, out_v, w_stage_v, w_sh,
         sem_d, sem_a, sem_b, gsem_a, gsem_b) = refs[F:]
        cid = lax.axis_index("c")
        sid = lax.axis_index("s")
        wid = sid * NC + cid
        rows = pl.ds(wid * BPW, BPW)
        FH = F // 2
        NH = FH * BPW
        c_dense = pltpu.async_copy(dense_hbm.at[rows], dense_v, sem_d)
        c_head = pltpu.async_copy(w_hbm.at[pl.ds(0, L)], whead_v, sem_d)
        half_a = [pltpu.async_copy(
            sparse_hbm[i].at[rows], idx_v.at[pl.ds(i * BPW, BPW)], sem_a)
            for i in range(FH)]
        half_b = [pltpu.async_copy(
            sparse_hbm[i].at[rows], idx_v.at[pl.ds(i * BPW, BPW)], sem_b)
            for i in range(FH, F)]
        for k in range(NS):
            size = min(NSLC, W_LEN - k * NSLC)
            @pl.when(sid == k)
            def _():
                pltpu.sync_copy(w_hbm.at[pl.ds(k * NSLC, size)],
                                w_stage_v.at[pl.ds(0, size)])
                pltpu.sync_copy(w_stage_v.at[pl.ds(0, size)],
                                w_sh.at[pl.ds(k * NSLC, size)])
        plsc.subcore_barrier()
        for c in half_a:
            c.wait()
        for i in range(FH):
            off = D + i * V
            for g in range(G):
                sl = pl.ds(i * BPW + g * L, L)
                idx_v[sl] = idx_v[sl] + off
        c_gath_a = pltpu.async_copy(
            w_sh.at[idx_v.at[pl.ds(0, NH)]], gath_v.at[pl.ds(0, NH)], gsem_a)
        for c in half_b:
            c.wait()
        for i in range(FH, F):
            off = D + i * V
            for g in range(G):
                sl = pl.ds(i * BPW + g * L, L)
                idx_v[sl] = idx_v[sl] + off
        c_gath_b = pltpu.async_copy(
            w_sh.at[idx_v.at[pl.ds(NH, NH)]], gath_v.at[pl.ds(NH, NH)],
            gsem_b)
        c_dense.wait()
        c_head.wait()
        w016 = whead_v[...]
        zero16 = lax.iota(jnp.int32, L) * 0
        for g in range(G):
            sl = pl.ds(g * L, L)
            rvec = lax.iota(jnp.int32, L) + (g * L)
            acc = plsc.load_gather(dense_v, [rvec, zero16]) * w016[0]
            for d in range(1, D):
                acc = acc + (plsc.load_gather(dense_v, [rvec, zero16 + d])
                             * w016[d])
            out_v[sl] = acc
        c_gath_a.wait()
        for g in range(G):
            sl = pl.ds(g * L, L)
            acc = out_v[sl]
            for i in range(FH):
                acc = acc + gath_v[pl.ds(i * BPW + g * L, L)]
            out_v[sl] = acc
        c_gath_b.wait()
        for g in range(G):
            sl = pl.ds(g * L, L)
            acc = out_v[sl]
            for i in range(FH, F):
                acc = acc + gath_v[pl.ds(i * BPW + g * L, L)]
            out_v[sl] = acc
        pltpu.sync_copy(out_v, out_hbm.at[rows])

    return body(dense_features, w_flat, *sparse)


def kernel(dense_features, W,
           sparse_0, sparse_1, sparse_2, sparse_3, sparse_4, sparse_5,
           sparse_6, sparse_7, sparse_8, sparse_9, sparse_10, sparse_11,
           sparse_12, sparse_13, sparse_14, sparse_15, sparse_16, sparse_17,
           sparse_18, sparse_19, sparse_20, sparse_21, sparse_22, sparse_23,
           sparse_24, sparse_25):
    sparse = [sparse_0, sparse_1, sparse_2, sparse_3, sparse_4, sparse_5,
              sparse_6, sparse_7, sparse_8, sparse_9, sparse_10, sparse_11,
              sparse_12, sparse_13, sparse_14, sparse_15, sparse_16,
              sparse_17, sparse_18, sparse_19, sparse_20, sparse_21,
              sparse_22, sparse_23, sparse_24, sparse_25]
    out = _wide_sc(dense_features, W[:, 0], sparse)
    return out[:, None]


# speedup vs baseline: 1.4174x; 1.0028x over previous
import functools

import jax
import jax.numpy as jnp
from jax import lax
from jax.experimental import pallas as pl
from jax.experimental.pallas import tpu as pltpu
from jax.experimental.pallas import tpu_sc as plsc

B = 4096
F = 26
V = 1000
D = 13
W_LEN = F * V + D

NC = 2
NS = 16
NW = NC * NS
BPW = B // NW
L = 16
G = BPW // L
NIDX = F * BPW
NSLC = 1632
W_IMG = NS * NSLC
NQ = 4
QB = (0, 7, 13, 20, F)


def _wide_sc(dense_features, w_flat, sparse):
    mesh = plsc.VectorSubcoreMesh(core_axis_name="c", subcore_axis_name="s")

    @functools.partial(
        pl.kernel,
        mesh=mesh,
        out_type=jax.ShapeDtypeStruct((B,), jnp.float32),
        compiler_params=pltpu.CompilerParams(needs_layout_passes=False),
        scratch_types=[
            pltpu.VMEM((NIDX,), jnp.int32),
            pltpu.VMEM((BPW, D), jnp.float32),
            pltpu.VMEM((L,), jnp.float32),
            pltpu.VMEM((NIDX,), jnp.float32),
            pltpu.VMEM((BPW,), jnp.float32),
            pltpu.VMEM((NSLC,), jnp.float32),
            pltpu.VMEM_SHARED((W_IMG,), jnp.float32),
            pltpu.SemaphoreType.DMA,
        ] + [pltpu.SemaphoreType.DMA] * (2 * NQ),
    )
    def body(dense_hbm, w_hbm, *refs):
        sparse_hbm = refs[:F]
        (out_hbm, idx_v, dense_v, whead_v, gath_v, out_v, w_stage_v, w_sh,
         sem_d) = refs[F:F + 9]
        isems = refs[F + 9:F + 9 + NQ]
        gsems = refs[F + 9 + NQ:]
        cid = lax.axis_index("c")
        sid = lax.axis_index("s")
        wid = sid * NC + cid
        rows = pl.ds(wid * BPW, BPW)
        c_dense = pltpu.async_copy(dense_hbm.at[rows], dense_v, sem_d)
        c_head = pltpu.async_copy(w_hbm.at[pl.ds(0, L)], whead_v, sem_d)
        qcopies = [
            [pltpu.async_copy(
                sparse_hbm[i].at[rows], idx_v.at[pl.ds(i * BPW, BPW)],
                isems[q])
             for i in range(QB[q], QB[q + 1])]
            for q in range(NQ)]
        for k in range(NS):
            size = min(NSLC, W_LEN - k * NSLC)
            @pl.when(sid == k)
            def _():
                pltpu.sync_copy(w_hbm.at[pl.ds(k * NSLC, size)],
                                w_stage_v.at[pl.ds(0, size)])
                pltpu.sync_copy(w_stage_v.at[pl.ds(0, size)],
                                w_sh.at[pl.ds(k * NSLC, size)])
        plsc.subcore_barrier()
        gath_copies = []
        for q in range(NQ):
            for c in qcopies[q]:
                c.wait()
            for i in range(QB[q], QB[q + 1]):
                off = D + i * V
                for g in range(G):
                    sl = pl.ds(i * BPW + g * L, L)
                    idx_v[sl] = idx_v[sl] + off
            lo = QB[q] * BPW
            n = (QB[q + 1] - QB[q]) * BPW
            gath_copies.append(pltpu.async_copy(
                w_sh.at[idx_v.at[pl.ds(lo, n)]], gath_v.at[pl.ds(lo, n)],
                gsems[q]))
        c_dense.wait()
        c_head.wait()
        w016 = whead_v[...]
        zero16 = lax.iota(jnp.int32, L) * 0
        for g in range(G):
            sl = pl.ds(g * L, L)
            rvec = lax.iota(jnp.int32, L) + (g * L)
            acc = plsc.load_gather(dense_v, [rvec, zero16]) * w016[0]
            for d in range(1, D):
                acc = acc + (plsc.load_gather(dense_v, [rvec, zero16 + d])
                             * w016[d])
            out_v[sl] = acc
        for q in range(NQ):
            gath_copies[q].wait()
            for g in range(G):
                sl = pl.ds(g * L, L)
                acc = out_v[sl]
                for i in range(QB[q], QB[q + 1]):
                    acc = acc + gath_v[pl.ds(i * BPW + g * L, L)]
                out_v[sl] = acc
        pltpu.sync_copy(out_v, out_hbm.at[rows])

    return body(dense_features, w_flat, *sparse)


def kernel(dense_features, W,
           sparse_0, sparse_1, sparse_2, sparse_3, sparse_4, sparse_5,
           sparse_6, sparse_7, sparse_8, sparse_9, sparse_10, sparse_11,
           sparse_12, sparse_13, sparse_14, sparse_15, sparse_16, sparse_17,
           sparse_18, sparse_19, sparse_20, sparse_21, sparse_22, sparse_23,
           sparse_24, sparse_25):
    sparse = [sparse_0, sparse_1, sparse_2, sparse_3, sparse_4, sparse_5,
              sparse_6, sparse_7, sparse_8, sparse_9, sparse_10, sparse_11,
              sparse_12, sparse_13, sparse_14, sparse_15, sparse_16,
              sparse_17, sparse_18, sparse_19, sparse_20, sparse_21,
              sparse_22, sparse_23, sparse_24, sparse_25]
    out = _wide_sc(dense_features, W[:, 0], sparse)
    return out[:, None]
